# Initial kernel scaffold; baseline (speedup 1.0000x reference)
import jax
import jax.numpy as jnp
from jax.experimental import pallas as pl


def kernel(x, edge_index, W_l, W_r, b_l):
    raise NotImplementedError("write your pallas kernel here")


# speedup vs baseline: 3.8985x; 3.8985x over previous
import jax
import jax.numpy as jnp
from jax import lax
from jax.experimental import pallas as pl
from jax.experimental.pallas import tpu as pltpu
from jax.experimental.pallas import tpu_sc as plsc

_N = 10000
_E = 160000
_D = 256
_Q = 64
_CHUNK = 80
_JCH = 125
_RPT = 640
_NPAD = 10240


def _sc_agg_kernel(x0, x1, x2, x3, src3d, dst3d, zrow, zdeg,
                   agg0, agg1, agg2, agg3, deg_out,
                   idx_src, idx_dst, rows, obuf, ones_v, dbuf, sem,
                   acc_sh, deg_sh):
    c = lax.axis_index("c")
    s = lax.axis_index("s")

    pltpu.sync_copy(src3d.at[s], idx_src)
    pltpu.sync_copy(dst3d.at[s], idx_dst)

    def one_pass(x_tab, agg_out, with_deg):
        pltpu.sync_copy(zrow, acc_sh.at[pl.ds(s * _RPT, _RPT)])
        if with_deg:
            pltpu.sync_copy(zdeg.at[pl.ds(s * _RPT, _RPT)],
                            deg_sh.at[pl.ds(s * _RPT, _RPT)])
            for k in range(_CHUNK // 16):
                ones_v[pl.ds(16 * k, 16)] = jnp.ones((16,), jnp.float32)
        plsc.subcore_barrier()

        def body(j, carry):
            pltpu.async_copy(x_tab.at[idx_src.at[j]], rows, sem).wait()
            pltpu.sync_copy(rows, acc_sh.at[idx_dst.at[j]], add=True)
            if with_deg:
                pltpu.sync_copy(ones_v, deg_sh.at[idx_dst.at[j]], add=True)
            return carry

        lax.fori_loop(0, _JCH, body, 0)
        plsc.subcore_barrier()

        for k in range(5):
            r0 = s * _RPT + k * 128
            pltpu.sync_copy(acc_sh.at[pl.ds(r0, 128)], obuf)
            pltpu.sync_copy(obuf, agg_out.at[pl.ds(r0, 128)])
        if with_deg:
            pltpu.sync_copy(deg_sh.at[pl.ds(s * _RPT, _RPT)], dbuf)
            pltpu.sync_copy(dbuf, deg_out.at[pl.ds(s * _RPT, _RPT)])
        plsc.subcore_barrier()

    @pl.when(c == 0)
    def _():
        one_pass(x0, agg0, True)
        one_pass(x1, agg1, False)

    @pl.when(c == 1)
    def _():
        one_pass(x2, agg2, False)
        one_pass(x3, agg3, False)


_sc_agg = pl.kernel(
    _sc_agg_kernel,
    out_type=[
        jax.ShapeDtypeStruct((_NPAD, _Q), jnp.float32),
        jax.ShapeDtypeStruct((_NPAD, _Q), jnp.float32),
        jax.ShapeDtypeStruct((_NPAD, _Q), jnp.float32),
        jax.ShapeDtypeStruct((_NPAD, _Q), jnp.float32),
        jax.ShapeDtypeStruct((_NPAD,), jnp.float32),
    ],
    mesh=plsc.VectorSubcoreMesh(core_axis_name="c", subcore_axis_name="s"),
    compiler_params=pltpu.CompilerParams(use_tc_tiling_on_sc=False),
    scratch_types=[
        pltpu.VMEM((_JCH, _CHUNK), jnp.int32),
        pltpu.VMEM((_JCH, _CHUNK), jnp.int32),
        pltpu.VMEM((_CHUNK, _Q), jnp.float32),
        pltpu.VMEM((128, _Q), jnp.float32),
        pltpu.VMEM((_CHUNK,), jnp.float32),
        pltpu.VMEM((_RPT,), jnp.float32),
        pltpu.SemaphoreType.DMA,
        pltpu.VMEM_SHARED((_NPAD, _Q), jnp.float32),
        pltpu.VMEM_SHARED((_NPAD,), jnp.float32),
    ],
)


_BLK = 1000


def _tc_dense_kernel(x_ref, a0_ref, a1_ref, a2_ref, a3_ref, deg_ref,
                     wl_ref, wr_ref, b_ref, o_ref):
    agg = jnp.concatenate(
        [a0_ref[...], a1_ref[...], a2_ref[...], a3_ref[...]], axis=1)
    deg = jnp.maximum(deg_ref[...], 1.0)
    mean = agg / deg
    h = lax.dot_general(mean, wl_ref[...], (((1,), (1,)), ((), ())),
                        preferred_element_type=jnp.float32)
    h = h + b_ref[...]
    h = h + lax.dot_general(x_ref[...], wr_ref[...], (((1,), (1,)), ((), ())),
                            preferred_element_type=jnp.float32)
    ss = jnp.sum(h * h, axis=1, keepdims=True)
    o_ref[...] = h / jnp.maximum(jnp.sqrt(ss), 1e-12)


def _tc_dense(x, a0, a1, a2, a3, deg, W_l, W_r, b_l):
    grid = (_N // _BLK,)
    qspec = pl.BlockSpec((_BLK, _Q), lambda i: (i, 0))
    return pl.pallas_call(
        _tc_dense_kernel,
        grid=grid,
        in_specs=[
            pl.BlockSpec((_BLK, _D), lambda i: (i, 0)),
            qspec, qspec, qspec, qspec,
            pl.BlockSpec((_BLK, 1), lambda i: (i, 0)),
            pl.BlockSpec((_D, _D), lambda i: (0, 0)),
            pl.BlockSpec((_D, _D), lambda i: (0, 0)),
            pl.BlockSpec((1, _D), lambda i: (0, 0)),
        ],
        out_specs=pl.BlockSpec((_BLK, _D), lambda i: (i, 0)),
        out_shape=jax.ShapeDtypeStruct((_N, _D), jnp.float32),
    )(x, a0, a1, a2, a3, deg, W_l, W_r, b_l)


@jax.jit
def kernel(x, edge_index, W_l, W_r, b_l):
    src = edge_index[0].astype(jnp.int32).reshape(16, _JCH, _CHUNK)
    dst = edge_index[1].astype(jnp.int32).reshape(16, _JCH, _CHUNK)
    xq = [x[:, i * _Q:(i + 1) * _Q] for i in range(4)]
    zrow = jnp.zeros((_RPT, _Q), jnp.float32)
    zdeg = jnp.zeros((_NPAD,), jnp.float32)
    a0, a1, a2, a3, deg = _sc_agg(xq[0], xq[1], xq[2], xq[3],
                                  src, dst, zrow, zdeg)
    degc = deg[:_N].reshape(_N, 1)
    return _tc_dense(x, a0[:_N], a1[:_N], a2[:_N], a3[:_N], degc,
                     W_l, W_r, b_l.reshape(1, _D))


# speedup vs baseline: 7.2098x; 1.8494x over previous
import jax
import jax.numpy as jnp
from jax import lax
from jax.experimental import pallas as pl
from jax.experimental.pallas import tpu as pltpu
from jax.experimental.pallas import tpu_sc as plsc

_N = 10000
_E = 160000
_D = 256
_H = 128
_CHUNK = 80
_JCH = 125
_RPT = 640
_NPAD = 10240


def _sc_agg_kernel(x_lo, x_hi, src3d, dst3d, zrow, zdeg,
                   agg_lo, agg_hi, deg_out,
                   idx_src, idx_dst, rows, ones_v, dbuf,
                   sga, sgb, ssa, ssb, sd,
                   acc_sh, deg_sh):
    c = lax.axis_index("c")
    s = lax.axis_index("s")

    pltpu.sync_copy(src3d.at[s], idx_src)
    pltpu.sync_copy(dst3d.at[s], idx_dst)

    def run(x_tab, agg_out, with_deg):
        pltpu.sync_copy(zrow, acc_sh.at[pl.ds(s * _RPT, _RPT)])
        if with_deg:
            pltpu.sync_copy(zdeg.at[pl.ds(s * _RPT, _RPT)],
                            deg_sh.at[pl.ds(s * _RPT, _RPT)])
            for k in range(_CHUNK // 16):
                ones_v[pl.ds(16 * k, 16)] = jnp.ones((16,), jnp.float32)
        plsc.subcore_barrier()

        def gather(j, buf, sem):
            pltpu.async_copy(x_tab.at[idx_src.at[j]], rows.at[buf], sem)

        def gather_wait(j, buf, sem):
            pltpu.make_async_copy(x_tab.at[idx_src.at[j]], rows.at[buf],
                                  sem).wait()

        def scatter(j, buf, sem):
            pltpu.async_copy(rows.at[buf], acc_sh.at[idx_dst.at[j]], sem,
                             add=True).wait()

        def scatter_wait(j, buf, sem):
            pass

        def deg_add(j):
            if with_deg:
                pltpu.async_copy(ones_v, deg_sh.at[idx_dst.at[j]], sd,
                                 add=True).wait()

        def deg_wait(j):
            pass

        gather(0, 0, sga)

        def body(i, carry):
            a = 2 * i
            gather_wait(a, 0, sga)

            @pl.when(i > 0)
            def _():
                scatter_wait(a - 1, 1, ssb)
                deg_wait(a - 1)
            gather(a + 1, 1, sgb)
            scatter(a, 0, ssa)
            deg_add(a)
            gather_wait(a + 1, 1, sgb)
            scatter_wait(a, 0, ssa)
            deg_wait(a)
            gather(a + 2, 0, sga)
            scatter(a + 1, 1, ssb)
            deg_add(a + 1)
            return carry

        lax.fori_loop(0, (_JCH - 1) // 2, body, 0)
        j_last = _JCH - 1
        gather_wait(j_last, 0, sga)
        scatter_wait(j_last - 1, 1, ssb)
        deg_wait(j_last - 1)
        scatter(j_last, 0, ssa)
        deg_add(j_last)
        scatter_wait(j_last, 0, ssa)
        deg_wait(j_last)
        plsc.subcore_barrier()

        pltpu.sync_copy(acc_sh.at[pl.ds(s * _RPT, _CHUNK)], rows.at[0])
        for k in range(8):
            b = k % 2
            if k < 7:
                r1 = s * _RPT + (k + 1) * _CHUNK
                pltpu.async_copy(acc_sh.at[pl.ds(r1, _CHUNK)],
                                 rows.at[1 - b], sga)
            r0 = s * _RPT + k * _CHUNK
            pltpu.sync_copy(rows.at[b], agg_out.at[pl.ds(r0, _CHUNK)])
            if k < 7:
                r1 = s * _RPT + (k + 1) * _CHUNK
                pltpu.make_async_copy(acc_sh.at[pl.ds(r1, _CHUNK)],
                                      rows.at[1 - b], sga).wait()
        if with_deg:
            pltpu.sync_copy(deg_sh.at[pl.ds(s * _RPT, _RPT)], dbuf)
            pltpu.sync_copy(dbuf, deg_out.at[pl.ds(s * _RPT, _RPT)])

    @pl.when(c == 0)
    def _():
        run(x_lo, agg_lo, True)

    @pl.when(c == 1)
    def _():
        run(x_hi, agg_hi, False)


_sc_agg = pl.kernel(
    _sc_agg_kernel,
    out_type=[
        jax.ShapeDtypeStruct((_NPAD, _H), jnp.float32),
        jax.ShapeDtypeStruct((_NPAD, _H), jnp.float32),
        jax.ShapeDtypeStruct((_NPAD,), jnp.float32),
    ],
    mesh=plsc.VectorSubcoreMesh(core_axis_name="c", subcore_axis_name="s"),
    compiler_params=pltpu.CompilerParams(use_tc_tiling_on_sc=False),
    scratch_types=[
        pltpu.VMEM((_JCH, _CHUNK), jnp.int32),
        pltpu.VMEM((_JCH, _CHUNK), jnp.int32),
        pltpu.VMEM((2, _CHUNK, _H), jnp.float32),
        pltpu.VMEM((_CHUNK,), jnp.float32),
        pltpu.VMEM((_RPT,), jnp.float32),
        pltpu.SemaphoreType.DMA,
        pltpu.SemaphoreType.DMA,
        pltpu.SemaphoreType.DMA,
        pltpu.SemaphoreType.DMA,
        pltpu.SemaphoreType.DMA,
        pltpu.VMEM_SHARED((_NPAD, _H), jnp.float32),
        pltpu.VMEM_SHARED((_NPAD,), jnp.float32),
    ],
)


_BLK = 1000


def _tc_dense_kernel(x_ref, alo_ref, ahi_ref, deg_ref,
                     wl_ref, wr_ref, b_ref, o_ref):
    agg = jnp.concatenate([alo_ref[...], ahi_ref[...]], axis=1)
    deg = jnp.maximum(deg_ref[...], 1.0)
    mean = agg / deg
    h = lax.dot_general(mean, wl_ref[...], (((1,), (1,)), ((), ())),
                        preferred_element_type=jnp.float32)
    h = h + b_ref[...]
    h = h + lax.dot_general(x_ref[...], wr_ref[...], (((1,), (1,)), ((), ())),
                            preferred_element_type=jnp.float32)
    ss = jnp.sum(h * h, axis=1, keepdims=True)
    o_ref[...] = h / jnp.maximum(jnp.sqrt(ss), 1e-12)


def _tc_dense(x, alo, ahi, deg, W_l, W_r, b_l):
    grid = (_N // _BLK,)
    hspec = pl.BlockSpec((_BLK, _H), lambda i: (i, 0))
    return pl.pallas_call(
        _tc_dense_kernel,
        grid=grid,
        in_specs=[
            pl.BlockSpec((_BLK, _D), lambda i: (i, 0)),
            hspec, hspec,
            pl.BlockSpec((_BLK, 1), lambda i: (i, 0)),
            pl.BlockSpec((_D, _D), lambda i: (0, 0)),
            pl.BlockSpec((_D, _D), lambda i: (0, 0)),
            pl.BlockSpec((1, _D), lambda i: (0, 0)),
        ],
        out_specs=pl.BlockSpec((_BLK, _D), lambda i: (i, 0)),
        out_shape=jax.ShapeDtypeStruct((_N, _D), jnp.float32),
    )(x, alo, ahi, deg, W_l, W_r, b_l)


@jax.jit
def kernel(x, edge_index, W_l, W_r, b_l):
    src = edge_index[0].astype(jnp.int32).reshape(16, _JCH, _CHUNK)
    dst = edge_index[1].astype(jnp.int32).reshape(16, _JCH, _CHUNK)
    x_lo = x[:, :_H]
    x_hi = x[:, _H:]
    zrow = jnp.zeros((_RPT, _H), jnp.float32)
    zdeg = jnp.zeros((_NPAD,), jnp.float32)
    alo, ahi, deg = _sc_agg(x_lo, x_hi, src, dst, zrow, zdeg)
    degc = deg[:_N].reshape(_N, 1)
    return _tc_dense(x, alo[:_N], ahi[:_N], degc,
                     W_l, W_r, b_l.reshape(1, _D))
